# Initial kernel scaffold; baseline (speedup 1.0000x reference)
import jax
import jax.numpy as jnp
from jax.experimental import pallas as pl


def kernel(x, edge_index, edge_attr, t_span, W_msg, b_msg, W_upd, b_upd):
    raise NotImplementedError("write your pallas kernel here")


# speedup vs baseline: 3.1932x; 3.1932x over previous
import functools

import jax
import jax.numpy as jnp
from jax import lax
from jax.experimental import pallas as pl
from jax.experimental.pallas import tpu as pltpu
from jax.experimental.pallas import tpu_sc as plsc

_NC = 2
_NS = 16
_C = 128
_BE = 2048
_BN = 2000


def _eproj_body(ea_ref, w3_ref, b_ref, o_ref, *, n_edges):
    i = pl.program_id(0)
    rows = jnp.dot(ea_ref[...], w3_ref[...],
                   preferred_element_type=jnp.float32) + b_ref[...]
    rid = i * _BE + lax.broadcasted_iota(jnp.int32, rows.shape, 0)
    o_ref[...] = jnp.where(rid < n_edges, rows, jnp.float32(-1e30))


def _eproj_call(ea, w3, b_m, n_edges, d):
    e_pad = ea.shape[0]
    return pl.pallas_call(
        functools.partial(_eproj_body, n_edges=n_edges),
        grid=(e_pad // _BE,),
        in_specs=[
            pl.BlockSpec((_BE, ea.shape[1]), lambda i: (i, 0)),
            pl.BlockSpec((ea.shape[1], d), lambda i: (0, 0)),
            pl.BlockSpec((1, d), lambda i: (0, 0)),
        ],
        out_specs=pl.BlockSpec((_BE, d), lambda i: (i, 0)),
        out_shape=jax.ShapeDtypeStruct((e_pad, d), jnp.float32),
    )(ea, w3, b_m)


def _proj_body(z_ref, w1_ref, w2_ref, xs_ref, xd_ref):
    z = z_ref[...]
    xs_ref[...] = jnp.dot(z, w1_ref[...], preferred_element_type=jnp.float32)
    xd_ref[...] = jnp.dot(z, w2_ref[...], preferred_element_type=jnp.float32)


def _proj_call(z, w1, w2):
    n, d = z.shape
    return pl.pallas_call(
        _proj_body,
        grid=(n // _BN,),
        in_specs=[
            pl.BlockSpec((_BN, d), lambda i: (i, 0)),
            pl.BlockSpec((d, d), lambda i: (0, 0)),
            pl.BlockSpec((d, d), lambda i: (0, 0)),
        ],
        out_specs=[
            pl.BlockSpec((_BN, d), lambda i: (i, 0)),
            pl.BlockSpec((_BN, d), lambda i: (i, 0)),
        ],
        out_shape=[
            jax.ShapeDtypeStruct((n, d), jnp.float32),
            jax.ShapeDtypeStruct((n, d), jnp.float32),
        ],
    )(z, w1, w2)


def _upd_body(p_ref, z_ref, agg_ref, y0_ref, ks_ref, u1_ref, u2_ref, b_ref,
              w1_ref, w2_ref, zn_ref, ksn_ref, xs_ref, xd_ref):
    a = p_ref[0]
    bb = p_ref[1]
    w = p_ref[2]
    r = p_ref[3]
    agg = agg_ref[0] + agg_ref[1]
    k = jnp.tanh(
        jnp.dot(z_ref[...], u1_ref[...], preferred_element_type=jnp.float32)
        + jnp.dot(agg, u2_ref[...], preferred_element_type=jnp.float32)
        + b_ref[...])
    ksn = r * ks_ref[...] + w * k
    zn = y0_ref[...] + a * k + bb * ksn
    zn_ref[...] = zn
    ksn_ref[...] = ksn
    xs_ref[...] = jnp.dot(zn, w1_ref[...], preferred_element_type=jnp.float32)
    xd_ref[...] = jnp.dot(zn, w2_ref[...], preferred_element_type=jnp.float32)


def _upd_call(params, z, agg, y0, ksum, u1, u2, b_u, w1, w2):
    n, d = z.shape
    node = jax.ShapeDtypeStruct((n, d), jnp.float32)
    return pl.pallas_call(
        _upd_body,
        grid=(n // _BN,),
        in_specs=[
            pl.BlockSpec(memory_space=pltpu.SMEM),
            pl.BlockSpec((_BN, d), lambda i: (i, 0)),
            pl.BlockSpec((_NC, _BN, d), lambda i: (0, i, 0)),
            pl.BlockSpec((_BN, d), lambda i: (i, 0)),
            pl.BlockSpec((_BN, d), lambda i: (i, 0)),
            pl.BlockSpec((d, d), lambda i: (0, 0)),
            pl.BlockSpec((d, d), lambda i: (0, 0)),
            pl.BlockSpec((1, d), lambda i: (0, 0)),
            pl.BlockSpec((d, d), lambda i: (0, 0)),
            pl.BlockSpec((d, d), lambda i: (0, 0)),
        ],
        out_specs=[pl.BlockSpec((_BN, d), lambda i: (i, 0))] * 4,
        out_shape=[node] * 4,
    )(params, z, agg, y0, ksum, u1, u2, b_u, w1, w2)


def _make_sc_agg(n_pad, d, e_pad):
    nw = _NC * _NS
    ept = e_pad // nw
    kpt = ept // _C
    npieces = n_pad // _C
    nv = d // 16

    mesh = plsc.VectorSubcoreMesh(core_axis_name="c", subcore_axis_name="s",
                                  num_cores=_NC, num_subcores=_NS)

    @functools.partial(
        pl.kernel,
        out_type=jax.ShapeDtypeStruct((_NC, n_pad, d), jnp.float32),
        mesh=mesh,
        scratch_types=[
            pltpu.VMEM_SHARED((n_pad, d), jnp.float32),
            pltpu.VMEM((_C,), jnp.int32),
            pltpu.VMEM((_C,), jnp.int32),
            pltpu.VMEM((_C, d), jnp.float32),
            pltpu.VMEM((_C, d), jnp.float32),
            pltpu.VMEM((_C, d), jnp.float32),
            pltpu.SemaphoreType.DMA,
            pltpu.SemaphoreType.DMA,
        ],
    )
    def sc_agg(src_hbm, dst_hbm, ep_hbm, xs_hbm, xd_hbm, out_hbm,
               agg_sh, src_v, dst_v, m_v, a_v, b_v, sem1, sem2):
        ci = lax.axis_index("c")
        si = lax.axis_index("s")
        wid = ci * _NS + si

        zvec = jnp.zeros((16,), jnp.float32)

        @pl.loop(0, _C)
        def _(rr):
            for j in range(nv):
                m_v[rr, pl.ds(j * 16, 16)] = zvec

        @pl.loop(si, npieces, step=_NS)
        def _(jz):
            pltpu.sync_copy(m_v, agg_sh.at[pl.ds(jz * _C, _C)])

        plsc.subcore_barrier()

        @pl.loop(0, kpt)
        def _(kk):
            base = wid * ept + kk * _C
            pltpu.sync_copy(src_hbm.at[pl.ds(base, _C)], src_v)
            pltpu.sync_copy(dst_hbm.at[pl.ds(base, _C)], dst_v)
            cp_e = pltpu.async_copy(ep_hbm.at[pl.ds(base, _C)], m_v, sem1)
            cp_s = pltpu.async_copy(xs_hbm.at[src_v], a_v, sem2)
            cp_d = pltpu.async_copy(xd_hbm.at[dst_v], b_v, sem2)
            cp_e.wait()
            cp_s.wait()
            cp_d.wait()

            @pl.loop(0, _C)
            def _(rr):
                for j in range(nv):
                    sl = pl.ds(j * 16, 16)
                    v = m_v[rr, sl] + a_v[rr, sl] + b_v[rr, sl]
                    m_v[rr, sl] = jnp.maximum(v, jnp.float32(0.0))

            pltpu.sync_copy(m_v, agg_sh.at[dst_v], add=True)

        plsc.subcore_barrier()

        @pl.loop(si, npieces, step=_NS)
        def _(jz):
            pltpu.sync_copy(agg_sh.at[pl.ds(jz * _C, _C)],
                            out_hbm.at[ci, pl.ds(jz * _C, _C)])

    return sc_agg


def kernel(x, edge_index, edge_attr, t_span, W_msg, b_msg, W_upd, b_upd):
    x0 = x[-1]
    n, d = x0.shape
    e = edge_index.shape[1]
    de = edge_attr.shape[1]
    t = t_span.shape[0]

    src = edge_index[0].astype(jnp.int32)
    dst = edge_index[1].astype(jnp.int32)
    nw_c = _NC * _NS * _C
    e_pad = ((e + nw_c - 1) // nw_c) * nw_c
    pad = e_pad - e
    ea = edge_attr
    if pad:
        src = jnp.concatenate([src, jnp.zeros((pad,), jnp.int32)])
        dst = jnp.concatenate([dst, jnp.zeros((pad,), jnp.int32)])
        ea = jnp.concatenate([ea, jnp.zeros((pad, de), ea.dtype)])

    w1 = W_msg[:d]
    w2 = W_msg[d:2 * d]
    w3 = W_msg[2 * d:]
    u1 = W_upd[:d]
    u2 = W_upd[d:]
    b_m = b_msg.reshape(1, d)
    b_u = b_upd.reshape(1, d)

    eproj = _eproj_call(ea, w3, b_m, e, d)
    xs, xd = _proj_call(x0, w1, w2)
    n_unit = _NS * 8
    n_pad = ((n + n_unit - 1) // n_unit) * n_unit
    n_pad = ((n_pad + _C - 1) // _C) * _C
    sc_agg = _make_sc_agg(n_pad, d, e_pad)

    y0 = x0
    z = x0
    ksum = jnp.zeros_like(x0)
    outs = []
    one = jnp.float32(1.0)
    two = jnp.float32(2.0)
    zero = jnp.float32(0.0)
    for i in range(t - 1):
        h = (t_span[i + 1] - t_span[i]).astype(jnp.float32)
        coeffs = [
            (h / 2, zero, one, zero),
            (h / 2, zero, two, one),
            (h, zero, two, one),
            (zero, h / 6, one, one),
        ]
        for a_, b_, w_, r_ in coeffs:
            agg = sc_agg(src, dst, eproj, xs, xd)
            params = jnp.stack([a_, b_, w_, r_])
            z, ksum, xs, xd = _upd_call(params, z, agg, y0, ksum,
                                        u1, u2, b_u, w1, w2)
        y0 = z
        outs.append(z)
    return jnp.stack(outs, axis=0)


# speedup vs baseline: 4.0688x; 1.2742x over previous
import functools

import jax
import jax.numpy as jnp
from jax import lax
from jax.experimental import pallas as pl
from jax.experimental.pallas import tpu as pltpu
from jax.experimental.pallas import tpu_sc as plsc

_NC = 2
_NS = 16
_C = 64
_BE = 2048
_BN = 2000


def _eproj_body(ea_ref, w3_ref, b_ref, o_ref, *, n_edges):
    i = pl.program_id(0)
    rows = jnp.dot(ea_ref[...], w3_ref[...],
                   preferred_element_type=jnp.float32) + b_ref[...]
    rid = i * _BE + lax.broadcasted_iota(jnp.int32, rows.shape, 0)
    o_ref[...] = jnp.where(rid < n_edges, rows, jnp.float32(-1e30))


def _eproj_call(ea, w3, b_m, n_edges, d):
    e_pad = ea.shape[0]
    return pl.pallas_call(
        functools.partial(_eproj_body, n_edges=n_edges),
        grid=(e_pad // _BE,),
        in_specs=[
            pl.BlockSpec((_BE, ea.shape[1]), lambda i: (i, 0)),
            pl.BlockSpec((ea.shape[1], d), lambda i: (0, 0)),
            pl.BlockSpec((1, d), lambda i: (0, 0)),
        ],
        out_specs=pl.BlockSpec((_BE, d), lambda i: (i, 0)),
        out_shape=jax.ShapeDtypeStruct((e_pad, d), jnp.float32),
    )(ea, w3, b_m)


def _proj_body(z_ref, w1_ref, w2_ref, xs_ref, xd_ref):
    z = z_ref[...]
    xs_ref[...] = jnp.dot(z, w1_ref[...], preferred_element_type=jnp.float32)
    xd_ref[...] = jnp.dot(z, w2_ref[...], preferred_element_type=jnp.float32)


def _proj_call(z, w1, w2):
    n, d = z.shape
    return pl.pallas_call(
        _proj_body,
        grid=(n // _BN,),
        in_specs=[
            pl.BlockSpec((_BN, d), lambda i: (i, 0)),
            pl.BlockSpec((d, d), lambda i: (0, 0)),
            pl.BlockSpec((d, d), lambda i: (0, 0)),
        ],
        out_specs=[
            pl.BlockSpec((_BN, d), lambda i: (i, 0)),
            pl.BlockSpec((_BN, d), lambda i: (i, 0)),
        ],
        out_shape=[
            jax.ShapeDtypeStruct((n, d), jnp.float32),
            jax.ShapeDtypeStruct((n, d), jnp.float32),
        ],
    )(z, w1, w2)


def _upd_body(p_ref, z_ref, agg_ref, y0_ref, ks_ref, u1_ref, u2_ref, b_ref,
              w1_ref, w2_ref, zn_ref, ksn_ref, xs_ref, xd_ref):
    a = p_ref[0]
    bb = p_ref[1]
    w = p_ref[2]
    r = p_ref[3]
    agg = agg_ref[0] + agg_ref[1]
    k = jnp.tanh(
        jnp.dot(z_ref[...], u1_ref[...], preferred_element_type=jnp.float32)
        + jnp.dot(agg, u2_ref[...], preferred_element_type=jnp.float32)
        + b_ref[...])
    ksn = r * ks_ref[...] + w * k
    zn = y0_ref[...] + a * k + bb * ksn
    zn_ref[...] = zn
    ksn_ref[...] = ksn
    xs_ref[...] = jnp.dot(zn, w1_ref[...], preferred_element_type=jnp.float32)
    xd_ref[...] = jnp.dot(zn, w2_ref[...], preferred_element_type=jnp.float32)


def _upd_call(params, z, agg, y0, ksum, u1, u2, b_u, w1, w2):
    n, d = z.shape
    node = jax.ShapeDtypeStruct((n, d), jnp.float32)
    return pl.pallas_call(
        _upd_body,
        grid=(n // _BN,),
        in_specs=[
            pl.BlockSpec(memory_space=pltpu.SMEM),
            pl.BlockSpec((_BN, d), lambda i: (i, 0)),
            pl.BlockSpec((_NC, _BN, d), lambda i: (0, i, 0)),
            pl.BlockSpec((_BN, d), lambda i: (i, 0)),
            pl.BlockSpec((_BN, d), lambda i: (i, 0)),
            pl.BlockSpec((d, d), lambda i: (0, 0)),
            pl.BlockSpec((d, d), lambda i: (0, 0)),
            pl.BlockSpec((1, d), lambda i: (0, 0)),
            pl.BlockSpec((d, d), lambda i: (0, 0)),
            pl.BlockSpec((d, d), lambda i: (0, 0)),
        ],
        out_specs=[pl.BlockSpec((_BN, d), lambda i: (i, 0))] * 4,
        out_shape=[node] * 4,
    )(params, z, agg, y0, ksum, u1, u2, b_u, w1, w2)


def _make_sc_agg(n_pad, d, e_pad):
    nw = _NC * _NS
    ept = e_pad // nw
    kpt = ept // _C
    npieces = n_pad // _C
    nv = d // 16

    mesh = plsc.VectorSubcoreMesh(core_axis_name="c", subcore_axis_name="s",
                                  num_cores=_NC, num_subcores=_NS)

    @functools.partial(
        pl.kernel,
        out_type=jax.ShapeDtypeStruct((_NC, n_pad, d), jnp.float32),
        mesh=mesh,
        scratch_types=[
            pltpu.VMEM_SHARED((n_pad, d), jnp.float32),
            pltpu.VMEM((2, 1, _C), jnp.int32),
            pltpu.VMEM((2, 1, _C), jnp.int32),
            pltpu.VMEM((2, _C, d), jnp.float32),
            pltpu.VMEM((2, _C, d), jnp.float32),
            pltpu.VMEM((2, _C, d), jnp.float32),
            pltpu.SemaphoreType.DMA,
            pltpu.SemaphoreType.DMA,
            pltpu.SemaphoreType.DMA,
        ],
    )
    def sc_agg(src_hbm, dst_hbm, ep_hbm, xs_hbm, xd_hbm, out_hbm,
               agg_sh, src_v, dst_v, m_v, a_v, b_v, sem_a, sem_b, sem_i):
        ci = lax.axis_index("c")
        si = lax.axis_index("s")
        wid = ci * _NS + si
        k_base = wid * kpt

        zvec = jnp.zeros((16,), jnp.float32)

        @pl.loop(0, _C)
        def _(rr):
            for j in range(nv):
                m_v[0, rr, pl.ds(j * 16, 16)] = zvec

        @pl.loop(si, npieces, step=_NS)
        def _(jz):
            pltpu.sync_copy(m_v.at[0], agg_sh.at[pl.ds(jz * _C, _C)])

        plsc.subcore_barrier()

        def issue_idx(k, ib):
            pltpu.async_copy(src_hbm.at[k_base + k], src_v.at[ib], sem_i)
            pltpu.async_copy(dst_hbm.at[k_base + k], dst_v.at[ib], sem_i)

        def wait_idx(ib):
            pltpu.make_async_copy(src_hbm.at[0], src_v.at[ib], sem_i).wait()
            pltpu.make_async_copy(src_hbm.at[0], dst_v.at[ib], sem_i).wait()

        def issue(k, buf, sem):
            pltpu.async_copy(ep_hbm.at[pl.ds((k_base + k) * _C, _C)],
                             m_v.at[buf], sem)
            pltpu.async_copy(xs_hbm.at[src_v.at[buf, 0]], a_v.at[buf], sem)
            pltpu.async_copy(xd_hbm.at[dst_v.at[buf, 0]], b_v.at[buf], sem)

        def process(buf, sem):
            pltpu.make_async_copy(ep_hbm.at[pl.ds(0, _C)], m_v.at[buf], sem).wait()
            pltpu.make_async_copy(ep_hbm.at[pl.ds(0, _C)], a_v.at[buf], sem).wait()
            pltpu.make_async_copy(ep_hbm.at[pl.ds(0, _C)], b_v.at[buf], sem).wait()

            @pl.loop(0, _C)
            def _(rr):
                for j in range(nv):
                    sl = pl.ds(j * 16, 16)
                    v = m_v[buf, rr, sl] + a_v[buf, rr, sl] + b_v[buf, rr, sl]
                    m_v[buf, rr, sl] = jnp.maximum(v, jnp.float32(0.0))

            pltpu.sync_copy(m_v.at[buf], agg_sh.at[dst_v.at[buf, 0]], add=True)

        pltpu.sync_copy(src_hbm.at[k_base], src_v.at[0])
        pltpu.sync_copy(dst_hbm.at[k_base], dst_v.at[0])
        issue(0, 0, sem_a)
        issue_idx(1, 1)

        @pl.loop(0, kpt // 2)
        def _(p):
            k0 = 2 * p
            wait_idx(1)
            issue(k0 + 1, 1, sem_b)
            process(0, sem_a)

            @pl.when(k0 + 2 < kpt)
            def _():
                issue_idx(k0 + 2, 0)

            process(1, sem_b)

            @pl.when(k0 + 2 < kpt)
            def _():
                wait_idx(0)
                issue(k0 + 2, 0, sem_a)

                @pl.when(k0 + 3 < kpt)
                def _():
                    issue_idx(k0 + 3, 1)

        plsc.subcore_barrier()

        @pl.loop(si, npieces, step=_NS)
        def _(jz):
            pltpu.sync_copy(agg_sh.at[pl.ds(jz * _C, _C)],
                            out_hbm.at[ci, pl.ds(jz * _C, _C)])

    return sc_agg


def kernel(x, edge_index, edge_attr, t_span, W_msg, b_msg, W_upd, b_upd):
    x0 = x[-1]
    n, d = x0.shape
    e = edge_index.shape[1]
    de = edge_attr.shape[1]
    t = t_span.shape[0]

    src = edge_index[0].astype(jnp.int32)
    dst = edge_index[1].astype(jnp.int32)
    nw_c = _NC * _NS * _C * 2
    e_pad = ((e + nw_c - 1) // nw_c) * nw_c
    pad = e_pad - e
    ea = edge_attr
    if pad:
        src = jnp.concatenate([src, jnp.zeros((pad,), jnp.int32)])
        dst = jnp.concatenate([dst, jnp.zeros((pad,), jnp.int32)])
        ea = jnp.concatenate([ea, jnp.zeros((pad, de), ea.dtype)])
    src = src.reshape(e_pad // _C, 1, _C)
    dst = dst.reshape(e_pad // _C, 1, _C)

    w1 = W_msg[:d]
    w2 = W_msg[d:2 * d]
    w3 = W_msg[2 * d:]
    u1 = W_upd[:d]
    u2 = W_upd[d:]
    b_m = b_msg.reshape(1, d)
    b_u = b_upd.reshape(1, d)

    eproj = _eproj_call(ea, w3, b_m, e, d)
    xs, xd = _proj_call(x0, w1, w2)
    n_unit = _NS * 8
    n_pad = ((n + n_unit - 1) // n_unit) * n_unit
    n_pad = ((n_pad + _C - 1) // _C) * _C
    sc_agg = _make_sc_agg(n_pad, d, e_pad)

    y0 = x0
    z = x0
    ksum = jnp.zeros_like(x0)
    outs = []
    one = jnp.float32(1.0)
    two = jnp.float32(2.0)
    zero = jnp.float32(0.0)
    for i in range(t - 1):
        h = (t_span[i + 1] - t_span[i]).astype(jnp.float32)
        coeffs = [
            (h / 2, zero, one, zero),
            (h / 2, zero, two, one),
            (h, zero, two, one),
            (zero, h / 6, one, one),
        ]
        for a_, b_, w_, r_ in coeffs:
            agg = sc_agg(src, dst, eproj, xs, xd)
            params = jnp.stack([a_, b_, w_, r_])
            z, ksum, xs, xd = _upd_call(params, z, agg, y0, ksum,
                                        u1, u2, b_u, w1, w2)
        y0 = z
        outs.append(z)
    return jnp.stack(outs, axis=0)


# speedup vs baseline: 7.3065x; 1.7958x over previous
import functools

import jax
import jax.numpy as jnp
from jax import lax
from jax.experimental import pallas as pl
from jax.experimental.pallas import tpu as pltpu
from jax.experimental.pallas import tpu_sc as plsc

_NC = 2
_NS = 16
_C = 32
_BE = 2048
_BN = 2000


def _eproj_body(ea_ref, w3_ref, b_ref, o_ref, *, n_edges):
    i = pl.program_id(0)
    rows = jnp.dot(ea_ref[...], w3_ref[...],
                   preferred_element_type=jnp.float32) + b_ref[...]
    rid = i * _BE + lax.broadcasted_iota(jnp.int32, rows.shape, 0)
    o_ref[...] = jnp.where(rid < n_edges, rows, jnp.float32(-1e30))


def _eproj_call(ea, w3, b_m, n_edges, d):
    e_pad = ea.shape[0]
    return pl.pallas_call(
        functools.partial(_eproj_body, n_edges=n_edges),
        grid=(e_pad // _BE,),
        in_specs=[
            pl.BlockSpec((_BE, ea.shape[1]), lambda i: (i, 0)),
            pl.BlockSpec((ea.shape[1], d), lambda i: (0, 0)),
            pl.BlockSpec((1, d), lambda i: (0, 0)),
        ],
        out_specs=pl.BlockSpec((_BE, d), lambda i: (i, 0)),
        out_shape=jax.ShapeDtypeStruct((e_pad, d), jnp.float32),
    )(ea, w3, b_m)


def _proj_body(z_ref, w1_ref, w2_ref, xs_ref, xd_ref):
    z = z_ref[...]
    xs_ref[...] = jnp.dot(z, w1_ref[...], preferred_element_type=jnp.float32)
    xd_ref[...] = jnp.dot(z, w2_ref[...], preferred_element_type=jnp.float32)


def _proj_call(z, w1, w2):
    n, d = z.shape
    return pl.pallas_call(
        _proj_body,
        grid=(n // _BN,),
        in_specs=[
            pl.BlockSpec((_BN, d), lambda i: (i, 0)),
            pl.BlockSpec((d, d), lambda i: (0, 0)),
            pl.BlockSpec((d, d), lambda i: (0, 0)),
        ],
        out_specs=[
            pl.BlockSpec((_BN, d), lambda i: (i, 0)),
            pl.BlockSpec((_BN, d), lambda i: (i, 0)),
        ],
        out_shape=[
            jax.ShapeDtypeStruct((n, d), jnp.float32),
            jax.ShapeDtypeStruct((n, d), jnp.float32),
        ],
    )(z, w1, w2)


def _upd_body(p_ref, z_ref, agg_ref, y0_ref, ks_ref, u1_ref, u2_ref, b_ref,
              w1_ref, w2_ref, zn_ref, ksn_ref, xs_ref, xd_ref):
    a = p_ref[0]
    bb = p_ref[1]
    w = p_ref[2]
    r = p_ref[3]
    agg = agg_ref[0] + agg_ref[1]
    k = jnp.tanh(
        jnp.dot(z_ref[...], u1_ref[...], preferred_element_type=jnp.float32)
        + jnp.dot(agg, u2_ref[...], preferred_element_type=jnp.float32)
        + b_ref[...])
    ksn = r * ks_ref[...] + w * k
    zn = y0_ref[...] + a * k + bb * ksn
    zn_ref[...] = zn
    ksn_ref[...] = ksn
    xs_ref[...] = jnp.dot(zn, w1_ref[...], preferred_element_type=jnp.float32)
    xd_ref[...] = jnp.dot(zn, w2_ref[...], preferred_element_type=jnp.float32)


def _upd_call(params, z, agg, y0, ksum, u1, u2, b_u, w1, w2):
    n, d = z.shape
    node = jax.ShapeDtypeStruct((n, d), jnp.float32)
    return pl.pallas_call(
        _upd_body,
        grid=(n // _BN,),
        in_specs=[
            pl.BlockSpec(memory_space=pltpu.SMEM),
            pl.BlockSpec((_BN, d), lambda i: (i, 0)),
            pl.BlockSpec((_NC, _BN, d), lambda i: (0, i, 0)),
            pl.BlockSpec((_BN, d), lambda i: (i, 0)),
            pl.BlockSpec((_BN, d), lambda i: (i, 0)),
            pl.BlockSpec((d, d), lambda i: (0, 0)),
            pl.BlockSpec((d, d), lambda i: (0, 0)),
            pl.BlockSpec((1, d), lambda i: (0, 0)),
            pl.BlockSpec((d, d), lambda i: (0, 0)),
            pl.BlockSpec((d, d), lambda i: (0, 0)),
        ],
        out_specs=[pl.BlockSpec((_BN, d), lambda i: (i, 0))] * 4,
        out_shape=[node] * 4,
    )(params, z, agg, y0, ksum, u1, u2, b_u, w1, w2)


_NB = 3
_NI = 6


def _make_sc_agg(n_pad, d, e_pad):
    nw = _NC * _NS
    ept = e_pad // nw
    kpt = ept // _C
    npieces = n_pad // _C
    nv = d // 16

    mesh = plsc.VectorSubcoreMesh(core_axis_name="c", subcore_axis_name="s",
                                  num_cores=_NC, num_subcores=_NS)

    @functools.partial(
        pl.kernel,
        out_type=jax.ShapeDtypeStruct((_NC, n_pad, d), jnp.float32),
        mesh=mesh,
        scratch_types=(
            [pltpu.VMEM_SHARED((n_pad, d), jnp.float32),
             pltpu.VMEM((_NI, 1, _C), jnp.int32),
             pltpu.VMEM((_NI, 1, _C), jnp.int32),
             pltpu.VMEM((_NB, _C, d), jnp.float32),
             pltpu.VMEM((_NB, _C, d), jnp.float32),
             pltpu.VMEM((_NB, _C, d), jnp.float32)]
            + [pltpu.SemaphoreType.DMA] * (2 * _NB + _NI)
        ),
    )
    def sc_agg(src_hbm, dst_hbm, ep_hbm, xs_hbm, xd_hbm, out_hbm,
               agg_sh, src_v, dst_v, m_v, a_v, b_v, *sems):
        semd = sems[:_NB]
        semc = sems[_NB:2 * _NB]
        semi = sems[2 * _NB:]
        ci = lax.axis_index("c")
        si = lax.axis_index("s")
        wid = ci * _NS + si
        k_base = wid * kpt

        zvec = jnp.zeros((16,), jnp.float32)

        @pl.loop(0, _C)
        def _(rr):
            for j in range(nv):
                m_v[0, rr, pl.ds(j * 16, 16)] = zvec

        @pl.loop(si, npieces, step=_NS)
        def _(jz):
            pltpu.sync_copy(m_v.at[0], agg_sh.at[pl.ds(jz * _C, _C)])

        plsc.subcore_barrier()

        def issue_idx(k, s):
            pltpu.async_copy(src_hbm.at[k_base + k], src_v.at[s], semi[s])
            pltpu.async_copy(dst_hbm.at[k_base + k], dst_v.at[s], semi[s])

        def wait_idx(s):
            pltpu.make_async_copy(src_hbm.at[0], src_v.at[s], semi[s]).wait()
            pltpu.make_async_copy(src_hbm.at[0], dst_v.at[s], semi[s]).wait()

        def issue_data(k, buf, s):
            pltpu.async_copy(ep_hbm.at[pl.ds((k_base + k) * _C, _C)],
                             m_v.at[buf], semd[buf])
            pltpu.async_copy(xs_hbm.at[src_v.at[s, 0]], a_v.at[buf], semd[buf])
            pltpu.async_copy(xd_hbm.at[dst_v.at[s, 0]], b_v.at[buf], semd[buf])

        def wait_scatter(buf):
            pltpu.make_async_copy(ep_hbm.at[pl.ds(0, _C)], m_v.at[buf],
                                  semc[buf]).wait()

        def compute_and_scatter(buf, s):
            pltpu.make_async_copy(ep_hbm.at[pl.ds(0, _C)], m_v.at[buf],
                                  semd[buf]).wait()
            pltpu.make_async_copy(ep_hbm.at[pl.ds(0, _C)], a_v.at[buf],
                                  semd[buf]).wait()
            pltpu.make_async_copy(ep_hbm.at[pl.ds(0, _C)], b_v.at[buf],
                                  semd[buf]).wait()

            @pl.loop(0, _C)
            def _(rr):
                for j in range(nv):
                    sl = pl.ds(j * 16, 16)
                    v = m_v[buf, rr, sl] + a_v[buf, rr, sl] + b_v[buf, rr, sl]
                    m_v[buf, rr, sl] = jnp.maximum(v, jnp.float32(0.0))

            pltpu.async_copy(m_v.at[buf], agg_sh.at[dst_v.at[s, 0]],
                             semc[buf], add=True)

        for j in range(2):
            pltpu.sync_copy(src_hbm.at[k_base + j], src_v.at[j])
            pltpu.sync_copy(dst_hbm.at[k_base + j], dst_v.at[j])
        for j in range(2, _NI):
            issue_idx(j, j)
        for j in range(2):
            issue_data(j, j, j)

        @pl.loop(0, kpt // _NI)
        def _(g):
            k0 = g * _NI
            for b in range(_NI):
                kk = k0 + b
                buf = b % _NB
                compute_and_scatter(buf, b)

                @pl.when(kk + 2 < kpt)
                def _(kk=kk, b=b):
                    buf2 = (b + 2) % _NB

                    @pl.when(kk > 0)
                    def _():
                        wait_scatter(buf2)

                    wait_idx((b + 2) % _NI)
                    issue_data(kk + 2, buf2, (b + 2) % _NI)

                @pl.when((kk > 0) & (kk + 5 < kpt))
                def _(kk=kk, b=b):
                    issue_idx(kk + 5, (b + 5) % _NI)

        for buf in range(_NB):
            wait_scatter(buf)

        plsc.subcore_barrier()

        @pl.loop(si, npieces, step=_NS)
        def _(jz):
            pltpu.sync_copy(agg_sh.at[pl.ds(jz * _C, _C)],
                            out_hbm.at[ci, pl.ds(jz * _C, _C)])

    return sc_agg


def kernel(x, edge_index, edge_attr, t_span, W_msg, b_msg, W_upd, b_upd):
    x0 = x[-1]
    n, d = x0.shape
    e = edge_index.shape[1]
    de = edge_attr.shape[1]
    t = t_span.shape[0]

    src = edge_index[0].astype(jnp.int32)
    dst = edge_index[1].astype(jnp.int32)
    nw_c = _NC * _NS * _C * _NI
    e_pad = ((e + nw_c - 1) // nw_c) * nw_c
    pad = e_pad - e
    ea = edge_attr
    if pad:
        fill = jnp.arange(pad, dtype=jnp.int32) % jnp.int32(n)
        src = jnp.concatenate([src, fill])
        dst = jnp.concatenate([dst, fill])
        ea = jnp.concatenate([ea, jnp.zeros((pad, de), ea.dtype)])
    src = src.reshape(e_pad // _C, 1, _C)
    dst = dst.reshape(e_pad // _C, 1, _C)

    w1 = W_msg[:d]
    w2 = W_msg[d:2 * d]
    w3 = W_msg[2 * d:]
    u1 = W_upd[:d]
    u2 = W_upd[d:]
    b_m = b_msg.reshape(1, d)
    b_u = b_upd.reshape(1, d)

    eproj = _eproj_call(ea, w3, b_m, e, d)
    xs, xd = _proj_call(x0, w1, w2)
    n_unit = _NS * 8
    n_pad = ((n + n_unit - 1) // n_unit) * n_unit
    n_pad = ((n_pad + _C - 1) // _C) * _C
    sc_agg = _make_sc_agg(n_pad, d, e_pad)

    y0 = x0
    z = x0
    ksum = jnp.zeros_like(x0)
    outs = []
    one = jnp.float32(1.0)
    two = jnp.float32(2.0)
    zero = jnp.float32(0.0)
    for i in range(t - 1):
        h = (t_span[i + 1] - t_span[i]).astype(jnp.float32)
        coeffs = [
            (h / 2, zero, one, zero),
            (h / 2, zero, two, one),
            (h, zero, two, one),
            (zero, h / 6, one, one),
        ]
        for a_, b_, w_, r_ in coeffs:
            agg = sc_agg(src, dst, eproj, xs, xd)
            params = jnp.stack([a_, b_, w_, r_])
            z, ksum, xs, xd = _upd_call(params, z, agg, y0, ksum,
                                        u1, u2, b_u, w1, w2)
        y0 = z
        outs.append(z)
    return jnp.stack(outs, axis=0)


# speedup vs baseline: 7.8439x; 1.0735x over previous
import functools

import jax
import jax.numpy as jnp
from jax import lax
from jax.experimental import pallas as pl
from jax.experimental.pallas import tpu as pltpu
from jax.experimental.pallas import tpu_sc as plsc

_NC = 2
_NS = 16
_C = 40
_P = 32
_BE = 2048
_BN = 2000


def _eproj_body(ea_ref, w3_ref, b_ref, o_ref, *, n_edges):
    i = pl.program_id(0)
    rows = jnp.dot(ea_ref[...], w3_ref[...],
                   preferred_element_type=jnp.float32) + b_ref[...]
    rid = i * _BE + lax.broadcasted_iota(jnp.int32, rows.shape, 0)
    o_ref[...] = jnp.where(rid < n_edges, rows, jnp.float32(-1e30))


def _eproj_call(ea, w3, b_m, n_edges, d):
    e_pad = ea.shape[0]
    return pl.pallas_call(
        functools.partial(_eproj_body, n_edges=n_edges),
        grid=(e_pad // _BE,),
        in_specs=[
            pl.BlockSpec((_BE, ea.shape[1]), lambda i: (i, 0)),
            pl.BlockSpec((ea.shape[1], d), lambda i: (0, 0)),
            pl.BlockSpec((1, d), lambda i: (0, 0)),
        ],
        out_specs=pl.BlockSpec((_BE, d), lambda i: (i, 0)),
        out_shape=jax.ShapeDtypeStruct((e_pad, d), jnp.float32),
    )(ea, w3, b_m)


def _proj_body(z_ref, w1_ref, w2_ref, xs_ref, xd_ref):
    z = z_ref[...]
    xs_ref[...] = jnp.dot(z, w1_ref[...], preferred_element_type=jnp.float32)
    xd_ref[...] = jnp.dot(z, w2_ref[...], preferred_element_type=jnp.float32)


def _proj_call(z, w1, w2):
    n, d = z.shape
    return pl.pallas_call(
        _proj_body,
        grid=(n // _BN,),
        in_specs=[
            pl.BlockSpec((_BN, d), lambda i: (i, 0)),
            pl.BlockSpec((d, d), lambda i: (0, 0)),
            pl.BlockSpec((d, d), lambda i: (0, 0)),
        ],
        out_specs=[
            pl.BlockSpec((_BN, d), lambda i: (i, 0)),
            pl.BlockSpec((_BN, d), lambda i: (i, 0)),
        ],
        out_shape=[
            jax.ShapeDtypeStruct((n, d), jnp.float32),
            jax.ShapeDtypeStruct((n, d), jnp.float32),
        ],
    )(z, w1, w2)


def _upd_body(p_ref, z_ref, agg_ref, y0_ref, ks_ref, u1_ref, u2_ref, b_ref,
              w1_ref, w2_ref, zn_ref, ksn_ref, xs_ref, xd_ref):
    a = p_ref[0]
    bb = p_ref[1]
    w = p_ref[2]
    r = p_ref[3]
    agg = agg_ref[0] + agg_ref[1]
    k = jnp.tanh(
        jnp.dot(z_ref[...], u1_ref[...], preferred_element_type=jnp.float32)
        + jnp.dot(agg, u2_ref[...], preferred_element_type=jnp.float32)
        + b_ref[...])
    ksn = r * ks_ref[...] + w * k
    zn = y0_ref[...] + a * k + bb * ksn
    zn_ref[...] = zn
    ksn_ref[...] = ksn
    xs_ref[...] = jnp.dot(zn, w1_ref[...], preferred_element_type=jnp.float32)
    xd_ref[...] = jnp.dot(zn, w2_ref[...], preferred_element_type=jnp.float32)


def _upd_call(params, z, agg, y0, ksum, u1, u2, b_u, w1, w2):
    n, d = z.shape
    node = jax.ShapeDtypeStruct((n, d), jnp.float32)
    return pl.pallas_call(
        _upd_body,
        grid=(n // _BN,),
        in_specs=[
            pl.BlockSpec(memory_space=pltpu.SMEM),
            pl.BlockSpec((_BN, d), lambda i: (i, 0)),
            pl.BlockSpec((_NC, _BN, d), lambda i: (0, i, 0)),
            pl.BlockSpec((_BN, d), lambda i: (i, 0)),
            pl.BlockSpec((_BN, d), lambda i: (i, 0)),
            pl.BlockSpec((d, d), lambda i: (0, 0)),
            pl.BlockSpec((d, d), lambda i: (0, 0)),
            pl.BlockSpec((1, d), lambda i: (0, 0)),
            pl.BlockSpec((d, d), lambda i: (0, 0)),
            pl.BlockSpec((d, d), lambda i: (0, 0)),
        ],
        out_specs=[pl.BlockSpec((_BN, d), lambda i: (i, 0))] * 4,
        out_shape=[node] * 4,
    )(params, z, agg, y0, ksum, u1, u2, b_u, w1, w2)


_NB = 3
_NI = 6


def _make_sc_agg(n_pad, d, e_pad):
    nw = _NC * _NS
    ept = e_pad // nw
    kpt = ept // _C
    npieces = n_pad // _P
    nv = d // 16

    mesh = plsc.VectorSubcoreMesh(core_axis_name="c", subcore_axis_name="s",
                                  num_cores=_NC, num_subcores=_NS)

    @functools.partial(
        pl.kernel,
        out_type=jax.ShapeDtypeStruct((_NC, n_pad, d), jnp.float32),
        mesh=mesh,
        scratch_types=(
            [pltpu.VMEM_SHARED((n_pad, d), jnp.float32),
             pltpu.VMEM((_NI, 1, _C), jnp.int32),
             pltpu.VMEM((_NI, 1, _C), jnp.int32),
             pltpu.VMEM((_NB, _C, d), jnp.float32),
             pltpu.VMEM((_NB, _C, d), jnp.float32),
             pltpu.VMEM((_NB, _C, d), jnp.float32)]
            + [pltpu.SemaphoreType.DMA] * (2 * _NB + _NI)
        ),
    )
    def sc_agg(src_hbm, dst_hbm, ep_hbm, xs_hbm, xd_hbm, out_hbm,
               agg_sh, src_v, dst_v, m_v, a_v, b_v, *sems):
        semd = sems[:_NB]
        semc = sems[_NB:2 * _NB]
        semi = sems[2 * _NB:]
        ci = lax.axis_index("c")
        si = lax.axis_index("s")
        wid = ci * _NS + si
        k_base = wid * kpt

        zvec = jnp.zeros((16,), jnp.float32)

        @pl.loop(0, _C)
        def _(rr):
            for j in range(nv):
                m_v[0, rr, pl.ds(j * 16, 16)] = zvec

        @pl.loop(si, npieces, step=_NS)
        def _(jz):
            pltpu.sync_copy(m_v.at[0, pl.ds(0, _P)], agg_sh.at[pl.ds(jz * _P, _P)])

        plsc.subcore_barrier()

        def issue_idx(k, s):
            pltpu.async_copy(src_hbm.at[k_base + k], src_v.at[s], semi[s])
            pltpu.async_copy(dst_hbm.at[k_base + k], dst_v.at[s], semi[s])

        def wait_idx(s):
            pltpu.make_async_copy(src_hbm.at[0], src_v.at[s], semi[s]).wait()
            pltpu.make_async_copy(src_hbm.at[0], dst_v.at[s], semi[s]).wait()

        def issue_data(k, buf, s):
            pltpu.async_copy(ep_hbm.at[pl.ds((k_base + k) * _C, _C)],
                             m_v.at[buf], semd[buf])
            pltpu.async_copy(xs_hbm.at[src_v.at[s, 0]], a_v.at[buf], semd[buf])
            pltpu.async_copy(xd_hbm.at[dst_v.at[s, 0]], b_v.at[buf], semd[buf])

        def wait_scatter(buf):
            pltpu.make_async_copy(ep_hbm.at[pl.ds(0, _C)], m_v.at[buf],
                                  semc[buf]).wait()

        def compute_and_scatter(buf, s):
            pltpu.make_async_copy(ep_hbm.at[pl.ds(0, _C)], m_v.at[buf],
                                  semd[buf]).wait()
            pltpu.make_async_copy(ep_hbm.at[pl.ds(0, _C)], a_v.at[buf],
                                  semd[buf]).wait()
            pltpu.make_async_copy(ep_hbm.at[pl.ds(0, _C)], b_v.at[buf],
                                  semd[buf]).wait()

            @pl.loop(0, _C)
            def _(rr):
                for j in range(nv):
                    sl = pl.ds(j * 16, 16)
                    v = m_v[buf, rr, sl] + a_v[buf, rr, sl] + b_v[buf, rr, sl]
                    m_v[buf, rr, sl] = jnp.maximum(v, jnp.float32(0.0))

            pltpu.async_copy(m_v.at[buf], agg_sh.at[dst_v.at[s, 0]],
                             semc[buf], add=True)

        for j in range(2):
            pltpu.sync_copy(src_hbm.at[k_base + j], src_v.at[j])
            pltpu.sync_copy(dst_hbm.at[k_base + j], dst_v.at[j])
        for j in range(2, _NI):
            issue_idx(j, j)
        for j in range(2):
            issue_data(j, j, j)

        @pl.loop(0, kpt // _NI)
        def _(g):
            k0 = g * _NI
            for b in range(_NI):
                kk = k0 + b
                buf = b % _NB
                compute_and_scatter(buf, b)

                @pl.when(kk + 2 < kpt)
                def _(kk=kk, b=b):
                    buf2 = (b + 2) % _NB

                    @pl.when(kk > 0)
                    def _():
                        wait_scatter(buf2)

                    wait_idx((b + 2) % _NI)
                    issue_data(kk + 2, buf2, (b + 2) % _NI)

                @pl.when((kk > 0) & (kk + 5 < kpt))
                def _(kk=kk, b=b):
                    issue_idx(kk + 5, (b + 5) % _NI)

        for buf in range(_NB):
            wait_scatter(buf)

        plsc.subcore_barrier()

        @pl.loop(si, npieces, step=_NS)
        def _(jz):
            pltpu.sync_copy(agg_sh.at[pl.ds(jz * _P, _P)],
                            out_hbm.at[ci, pl.ds(jz * _P, _P)])

    return sc_agg


def kernel(x, edge_index, edge_attr, t_span, W_msg, b_msg, W_upd, b_upd):
    x0 = x[-1]
    n, d = x0.shape
    e = edge_index.shape[1]
    de = edge_attr.shape[1]
    t = t_span.shape[0]

    src = edge_index[0].astype(jnp.int32)
    dst = edge_index[1].astype(jnp.int32)
    nw_c = _NC * _NS * _C * _NI
    e_pad = ((e + nw_c - 1) // nw_c) * nw_c
    pad = e_pad - e
    ea = edge_attr
    if pad:
        fill = jnp.arange(pad, dtype=jnp.int32) % jnp.int32(n)
        src = jnp.concatenate([src, fill])
        dst = jnp.concatenate([dst, fill])
        ea = jnp.concatenate([ea, jnp.zeros((pad, de), ea.dtype)])
    src = src.reshape(e_pad // _C, 1, _C)
    dst = dst.reshape(e_pad // _C, 1, _C)

    w1 = W_msg[:d]
    w2 = W_msg[d:2 * d]
    w3 = W_msg[2 * d:]
    u1 = W_upd[:d]
    u2 = W_upd[d:]
    b_m = b_msg.reshape(1, d)
    b_u = b_upd.reshape(1, d)

    eproj = _eproj_call(ea, w3, b_m, e, d)
    xs, xd = _proj_call(x0, w1, w2)
    n_unit = _NS * 8
    n_pad = ((n + n_unit - 1) // n_unit) * n_unit
    n_pad = ((n_pad + _C - 1) // _C) * _C
    sc_agg = _make_sc_agg(n_pad, d, e_pad)

    y0 = x0
    z = x0
    ksum = jnp.zeros_like(x0)
    outs = []
    one = jnp.float32(1.0)
    two = jnp.float32(2.0)
    zero = jnp.float32(0.0)
    for i in range(t - 1):
        h = (t_span[i + 1] - t_span[i]).astype(jnp.float32)
        coeffs = [
            (h / 2, zero, one, zero),
            (h / 2, zero, two, one),
            (h, zero, two, one),
            (zero, h / 6, one, one),
        ]
        for a_, b_, w_, r_ in coeffs:
            agg = sc_agg(src, dst, eproj, xs, xd)
            params = jnp.stack([a_, b_, w_, r_])
            z, ksum, xs, xd = _upd_call(params, z, agg, y0, ksum,
                                        u1, u2, b_u, w1, w2)
        y0 = z
        outs.append(z)
    return jnp.stack(outs, axis=0)
